# Initial kernel scaffold; baseline (speedup 1.0000x reference)
import jax
import jax.numpy as jnp
from jax.experimental import pallas as pl


def kernel(x, pos_table):
    raise NotImplementedError("write your pallas kernel here")


# speedup vs baseline: 4.7102x; 4.7102x over previous
import jax
import jax.numpy as jnp
from jax.experimental import pallas as pl


_BLOCK_S = 512


def _add_pos_kernel(x_ref, pos_ref, out_ref):
    out_ref[...] = x_ref[...] + pos_ref[...][:, None, :]


def kernel(x, pos_table):
    seq_len, batch, d_model = x.shape
    grid = (seq_len // _BLOCK_S,)
    return pl.pallas_call(
        _add_pos_kernel,
        grid=grid,
        in_specs=[
            pl.BlockSpec((_BLOCK_S, batch, d_model), lambda i: (i, 0, 0)),
            pl.BlockSpec((_BLOCK_S, d_model), lambda i: (i, 0)),
        ],
        out_specs=pl.BlockSpec((_BLOCK_S, batch, d_model), lambda i: (i, 0, 0)),
        out_shape=jax.ShapeDtypeStruct((seq_len, batch, d_model), x.dtype),
    )(x, pos_table[:seq_len])
